# baseline (speedup 1.0000x reference)
import jax
import jax.numpy as jnp
from jax.experimental import pallas as pl
from jax.experimental.pallas import tpu as pltpu

IMG = 28
C1, C2 = 4, 8
NCLS = 10
PW1 = 32
PW2 = 18
L1 = 27 * PW1 + IMG
L2 = 13 * PW2 + 14
M1 = L1 - PW1 - 1
M2 = L2 - PW2 - 1
FCH = 512
BT = 128


def _fwd_kernel(xr_ref, w1_ref, b1_ref, w2_ref, b2_ref, d1_ref,
                wf1_ref, bf1_ref, wf2_ref, bf2_ref, out_ref,
                xs_ref, c1_ref, m1_ref, xp2_ref, c2_ref, m2_ref, h_ref):
    f32 = jnp.float32
    bt = xr_ref.shape[0]

    xs_ref[...] = jnp.zeros((bt, PW1 * PW1), f32)
    for i in range(IMG):
        dst = (i + 2) * PW1 + 2
        xs_ref[:, dst:dst + IMG] = xr_ref[:, i * IMG:(i + 1) * IMG]

    for base in range(0, L1, 128):
        wd = min(128, L1 - base)
        acc = [jnp.zeros((bt, wd), f32) for _ in range(C1)]
        for ki in range(5):
            for kj in range(5):
                off = base + ki * PW1 + kj
                xs = xs_ref[:, off:off + wd]
                for co in range(C1):
                    acc[co] = acc[co] + xs * w1_ref[co * 25 + ki * 5 + kj]
        for co in range(C1):
            c1_ref[co, :, base:base + wd] = jnp.maximum(acc[co] + b1_ref[co], 0.0)

    for ci in range(C1):
        m1_ref[...] = jnp.maximum(
            jnp.maximum(c1_ref[ci, :, 0:M1], c1_ref[ci, :, 1:M1 + 1]),
            jnp.maximum(c1_ref[ci, :, PW1:M1 + PW1],
                        c1_ref[ci, :, PW1 + 1:M1 + PW1 + 1]))
        xp2_ref[ci] = jnp.dot(m1_ref[...], d1_ref[...],
                              preferred_element_type=f32)

    for cog in range(2):
        cos = tuple(range(cog * 4, cog * 4 + 4))
        for base in (0, 124):
            wd = 124
            acc = [jnp.zeros((bt, wd), f32) for _ in cos]
            for ci in range(C1):
                for ki in range(5):
                    for kj in range(5):
                        off = base + ki * PW2 + kj
                        xs = xp2_ref[ci, :, off:off + wd]
                        for n, co in enumerate(cos):
                            acc[n] = acc[n] + xs * w2_ref[(co * C1 + ci) * 25 + ki * 5 + kj]
            for n, co in enumerate(cos):
                c2_ref[co, :, base:base + wd] = jnp.maximum(acc[n] + b2_ref[co], 0.0)

    for co in range(C2):
        m2_ref[:, co * M2:(co + 1) * M2] = jnp.maximum(
            jnp.maximum(c2_ref[co, :, 0:M2], c2_ref[co, :, 1:M2 + 1]),
            jnp.maximum(c2_ref[co, :, PW2:M2 + PW2],
                        c2_ref[co, :, PW2 + 1:M2 + PW2 + 1]))

    h = jnp.dot(m2_ref[...], wf1_ref[...], preferred_element_type=f32)
    h_ref[...] = jnp.maximum(h + bf1_ref[...], 0.0)
    out_ref[...] = jnp.dot(h_ref[...], wf2_ref[...],
                           preferred_element_type=f32) + bf2_ref[...]


def _lsm_kernel(z_ref, o_ref):
    z = z_ref[...]
    mx = jnp.max(z, axis=0, keepdims=True)
    lse = jnp.log(jnp.sum(jnp.exp(z - mx), axis=0, keepdims=True)) + mx
    o_ref[...] = z - lse


def _round_up(a, b):
    return (a + b - 1) // b * b


@jax.jit
def _forward(x, w1, b1, w2, b2, d1, wf1x, bf1, wf2, bf2):
    f32 = jnp.float32
    B = x.shape[0]
    xr = x.astype(f32).reshape(B, IMG * IMG)
    wf1 = wf1x.reshape(C2 * M2, FCH)

    bt = min(_round_up(B, 8), BT)
    b_pad = _round_up(B, bt)
    if b_pad != B:
        xr = jnp.pad(xr, ((0, b_pad - B), (0, 0)))

    vmem = pl.BlockSpec(memory_space=pltpu.MemorySpace.VMEM)
    smem = pl.BlockSpec(memory_space=pltpu.MemorySpace.SMEM)

    logits = pl.pallas_call(
        _fwd_kernel,
        out_shape=jax.ShapeDtypeStruct((b_pad, NCLS), f32),
        grid=(b_pad // bt,),
        in_specs=[
            pl.BlockSpec((bt, IMG * IMG), lambda i: (i, 0)),
            smem, smem, smem, smem,
            vmem,
            vmem, vmem, vmem, vmem,
        ],
        out_specs=pl.BlockSpec((bt, NCLS), lambda i: (i, 0)),
        scratch_shapes=[
            pltpu.VMEM((bt, PW1 * PW1), f32),
            pltpu.VMEM((C1, bt, L1), f32),
            pltpu.VMEM((bt, M1), f32),
            pltpu.VMEM((C1, bt, PW2 * PW2), f32),
            pltpu.VMEM((C2, bt, L2), f32),
            pltpu.VMEM((bt, C2 * M2), f32),
            pltpu.VMEM((bt, FCH), f32),
        ],
        compiler_params=pltpu.CompilerParams(
            dimension_semantics=("parallel",)),
    )(xr, w1, b1, w2, b2, d1, wf1, bf1, wf2, bf2)

    logits = logits[:B]

    return pl.pallas_call(
        _lsm_kernel,
        out_shape=jax.ShapeDtypeStruct((B, NCLS), f32),
        in_specs=[vmem],
        out_specs=vmem,
    )(logits)


def kernel(x, w1, b1, w2, b2, d1, wf1x, bf1, wf2, bf2):
    return _forward(x, w1, b1, w2, b2, d1, wf1x, bf1, wf2, bf2)


# speedup vs baseline: 5.6471x; 5.6471x over previous
import numpy as np
import jax
import jax.numpy as jnp
from jax.experimental import pallas as pl
from jax.experimental.pallas import tpu as pltpu

IMG = 28
C1, C2 = 4, 8
NCLS = 10
PW1 = 32
PW2 = 18
L1 = 27 * PW1 + IMG
L2 = 13 * PW2 + 14
M1 = L1 - PW1 - 1
M2 = L2 - PW2 - 1
S1 = 896
S2 = 384
S3 = 256
N1 = C1 * S1
K2 = C1 * S2
N2 = C2 * S3
W1MAX = L1 + PW1 + 1
W2MAX = L2 + PW2 + 1
T1W = N1 - 34
T2W = N2 - 19
FCH = 512
BT = 128


def _shift_eye(k, nq, npp, qlo=None, qhi=None, pmax=None):
    q = np.arange(nq)[:, None]
    p = np.arange(npp)[None, :]
    m = (q - p) == k
    if qlo is not None:
        m &= (q >= qlo) & (q <= qhi)
    if pmax is not None:
        m &= p <= pmax
    return m.astype(np.float32)


_U1 = np.stack([_shift_eye(k, PW1, IMG, qlo=2, qhi=29) for k in range(5)])
_E1 = np.stack([_shift_eye(k, PW1, PW1, qlo=2, qhi=29, pmax=27)
                for k in range(5)])
_U2 = np.stack([_shift_eye(k, PW2, 14) for k in range(5)])
_E2 = np.stack([_shift_eye(k, PW2, PW2, pmax=13) for k in range(5)])


def _build_band_mats(w1, w2):
    f32 = jnp.float32
    v1 = jnp.einsum('ckj,jab->ckab', w1.reshape(C1, 5, 5),
                    jnp.asarray(_E1))
    w1m = jnp.einsum('kip,ckjq->ijcpq', jnp.asarray(_U1), v1,
                     preferred_element_type=f32)
    w1m = w1m.reshape(PW1 * PW1, N1)
    v2 = jnp.einsum('ockj,jab->ockab', w2.reshape(C2, C1, 5, 5),
                    jnp.asarray(_E2))
    w2m = jnp.einsum('kip,ockjq->cijopq', jnp.asarray(_U2), v2,
                     preferred_element_type=f32)
    w2m = w2m.reshape(C1, PW2 * PW2, C2, 14 * PW2)
    w2m = jnp.pad(w2m, ((0, 0), (0, S2 - PW2 * PW2),
                        (0, 0), (0, S3 - 14 * PW2)))
    return w1m, w2m.reshape(K2, N2)


def _fwd_kernel(xr_ref, w1m_ref, b1r_ref, w2m_ref, b2r_ref, d1_ref,
                wf1_ref, bf1_ref, wf2_ref, bf2_ref, out_ref,
                xs_ref, y1_ref, t1_ref, xp2_ref, y2_ref, t2_ref, h_ref):
    f32 = jnp.float32
    bt = xr_ref.shape[0]

    xs_ref[...] = jnp.zeros((bt, PW1 * PW1), f32)
    for i in range(IMG):
        dst = (i + 2) * PW1 + 2
        xs_ref[:, dst:dst + IMG] = xr_ref[:, i * IMG:(i + 1) * IMG]

    y1_ref[...] = jnp.maximum(
        jnp.dot(xs_ref[...], w1m_ref[...], preferred_element_type=f32)
        + b1r_ref[...], 0.0)

    t1_ref[:, 0:T1W] = jnp.maximum(
        jnp.maximum(y1_ref[:, 0:T1W], y1_ref[:, 1:T1W + 1]),
        jnp.maximum(y1_ref[:, PW1:T1W + PW1], y1_ref[:, PW1 + 1:T1W + PW1 + 1]))

    for ci in range(C1):
        xp2_ref[:, ci * S2:ci * S2 + PW2 * PW2] = jnp.dot(
            t1_ref[:, ci * S1:ci * S1 + M1], d1_ref[...],
            preferred_element_type=f32)
        xp2_ref[:, ci * S2 + PW2 * PW2:(ci + 1) * S2] = jnp.zeros(
            (bt, S2 - PW2 * PW2), f32)

    y2_ref[...] = jnp.maximum(
        jnp.dot(xp2_ref[...], w2m_ref[...], preferred_element_type=f32)
        + b2r_ref[...], 0.0)

    t2_ref[:, 0:T2W] = jnp.maximum(
        jnp.maximum(y2_ref[:, 0:T2W], y2_ref[:, 1:T2W + 1]),
        jnp.maximum(y2_ref[:, PW2:T2W + PW2], y2_ref[:, PW2 + 1:T2W + PW2 + 1]))
    t2_ref[:, T2W:N2] = jnp.zeros((bt, N2 - T2W), f32)

    h_ref[...] = jnp.maximum(
        jnp.dot(t2_ref[...], wf1_ref[...], preferred_element_type=f32)
        + bf1_ref[...], 0.0)
    out_ref[...] = jnp.dot(h_ref[...], wf2_ref[...],
                           preferred_element_type=f32) + bf2_ref[...]


def _lsm_kernel(z_ref, o_ref):
    z = z_ref[...]
    mx = jnp.max(z, axis=0, keepdims=True)
    lse = jnp.log(jnp.sum(jnp.exp(z - mx), axis=0, keepdims=True)) + mx
    o_ref[...] = z - lse


def _round_up(a, b):
    return (a + b - 1) // b * b


@jax.jit
def _forward(x, w1, b1, w2, b2, d1, wf1x, bf1, wf2, bf2):
    f32 = jnp.float32
    B = x.shape[0]
    xr = x.astype(f32).reshape(B, IMG * IMG)

    w1m, w2m = _build_band_mats(w1, w2)
    b1r = jnp.repeat(b1, S1).reshape(1, N1)
    b2r = jnp.repeat(b2, S3).reshape(1, N2)
    wf1 = jnp.pad(wf1x, ((0, 0), (0, S3 - M2), (0, 0))).reshape(N2, FCH)

    bt = min(_round_up(B, 8), BT)
    b_pad = _round_up(B, bt)
    if b_pad != B:
        xr = jnp.pad(xr, ((0, b_pad - B), (0, 0)))

    vmem = pl.BlockSpec(memory_space=pltpu.MemorySpace.VMEM)

    logits = pl.pallas_call(
        _fwd_kernel,
        out_shape=jax.ShapeDtypeStruct((b_pad, NCLS), f32),
        grid=(b_pad // bt,),
        in_specs=[
            pl.BlockSpec((bt, IMG * IMG), lambda i: (i, 0)),
            vmem, vmem, vmem, vmem,
            vmem,
            vmem, vmem, vmem, vmem,
        ],
        out_specs=pl.BlockSpec((bt, NCLS), lambda i: (i, 0)),
        scratch_shapes=[
            pltpu.VMEM((bt, PW1 * PW1), f32),
            pltpu.VMEM((bt, N1), f32),
            pltpu.VMEM((bt, N1), f32),
            pltpu.VMEM((bt, K2), f32),
            pltpu.VMEM((bt, N2), f32),
            pltpu.VMEM((bt, N2), f32),
            pltpu.VMEM((bt, FCH), f32),
        ],
        compiler_params=pltpu.CompilerParams(
            dimension_semantics=("parallel",)),
    )(xr, w1m, b1r, w2m, b2r, d1, wf1, bf1, wf2, bf2)

    logits = logits[:B]

    return pl.pallas_call(
        _lsm_kernel,
        out_shape=jax.ShapeDtypeStruct((B, NCLS), f32),
        in_specs=[vmem],
        out_specs=vmem,
    )(logits)


def kernel(x, w1, b1, w2, b2, d1, wf1x, bf1, wf2, bf2):
    return _forward(x, w1, b1, w2, b2, d1, wf1x, bf1, wf2, bf2)


# speedup vs baseline: 6.3327x; 1.1214x over previous
import numpy as np
import jax
import jax.numpy as jnp
from jax.experimental import pallas as pl
from jax.experimental.pallas import tpu as pltpu

IMG = 28
C1, C2 = 4, 8
NCLS = 10
PW1 = 32
PW2 = 18
L1 = 27 * PW1 + IMG
L2 = 13 * PW2 + 14
M1 = L1 - PW1 - 1
M2 = L2 - PW2 - 1
S1 = 896
S2 = 384
S3 = 256
N1 = C1 * S1
K2 = C1 * S2
N2 = C2 * S3
T1W = N1 - 34
T2W = N2 - 19
FCH = 512
BT = 256


def _shift_eye(k, nq, npp, qlo=None, qhi=None, pmax=None):
    q = np.arange(nq)[:, None]
    p = np.arange(npp)[None, :]
    m = (q - p) == k
    if qlo is not None:
        m &= (q >= qlo) & (q <= qhi)
    if pmax is not None:
        m &= p <= pmax
    return m.astype(np.float32)


_U1 = np.stack([_shift_eye(k, PW1, IMG, qlo=2, qhi=29) for k in range(5)])
_E1 = np.stack([_shift_eye(k, PW1, PW1, qlo=2, qhi=29, pmax=27)
                for k in range(5)])
_U2 = np.stack([_shift_eye(k, PW2, 14) for k in range(5)])
_E2 = np.stack([_shift_eye(k, PW2, PW2, pmax=13) for k in range(5)])


def _build_band_mats(w1, w2):
    f32 = jnp.float32
    v1 = jnp.einsum('ckj,jab->ckab', w1.reshape(C1, 5, 5),
                    jnp.asarray(_E1))
    w1m = jnp.einsum('kip,ckjq->ijcpq', jnp.asarray(_U1), v1,
                     preferred_element_type=f32)
    w1m = w1m.reshape(PW1 * PW1, N1)
    v2 = jnp.einsum('ockj,jab->ockab', w2.reshape(C2, C1, 5, 5),
                    jnp.asarray(_E2))
    w2m = jnp.einsum('kip,ockjq->cijopq', jnp.asarray(_U2), v2,
                     preferred_element_type=f32)
    w2m = w2m.reshape(C1, PW2 * PW2, C2, 14 * PW2)
    w2m = jnp.pad(w2m, ((0, 0), (0, S2 - PW2 * PW2),
                        (0, 0), (0, S3 - 14 * PW2)))
    return w1m, w2m.reshape(K2, N2)


def _fwd_kernel(xr_ref, w1m_ref, b1r_ref, w2m_ref, b2r_ref, d1_ref,
                wf1_ref, bf1_ref, wf2_ref, bf2_ref, out_ref,
                xs_ref, y1_ref, t1_ref, xp2_ref, y2_ref, t2_ref, h_ref):
    f32 = jnp.float32
    bf16 = jnp.bfloat16
    bt = xr_ref.shape[0]

    xs_ref[...] = jnp.zeros((bt, PW1 * PW1), bf16)
    for i in range(IMG):
        dst = (i + 2) * PW1 + 2
        xs_ref[:, dst:dst + IMG] = xr_ref[:, i * IMG:(i + 1) * IMG].astype(bf16)

    y1_ref[...] = jnp.maximum(
        jnp.dot(xs_ref[...], w1m_ref[...], preferred_element_type=f32)
        + b1r_ref[...], 0.0)

    t1_ref[:, 0:T1W] = jnp.maximum(
        jnp.maximum(y1_ref[:, 0:T1W], y1_ref[:, 1:T1W + 1]),
        jnp.maximum(y1_ref[:, PW1:T1W + PW1],
                    y1_ref[:, PW1 + 1:T1W + PW1 + 1])).astype(bf16)

    for ci in range(C1):
        xp2_ref[:, ci * S2:ci * S2 + PW2 * PW2] = jnp.dot(
            t1_ref[:, ci * S1:ci * S1 + M1], d1_ref[...],
            preferred_element_type=f32).astype(bf16)
        xp2_ref[:, ci * S2 + PW2 * PW2:(ci + 1) * S2] = jnp.zeros(
            (bt, S2 - PW2 * PW2), bf16)

    y2_ref[...] = jnp.maximum(
        jnp.dot(xp2_ref[...], w2m_ref[...], preferred_element_type=f32)
        + b2r_ref[...], 0.0)

    t2_ref[:, 0:T2W] = jnp.maximum(
        jnp.maximum(y2_ref[:, 0:T2W], y2_ref[:, 1:T2W + 1]),
        jnp.maximum(y2_ref[:, PW2:T2W + PW2],
                    y2_ref[:, PW2 + 1:T2W + PW2 + 1])).astype(bf16)
    t2_ref[:, T2W:N2] = jnp.zeros((bt, N2 - T2W), bf16)

    h_ref[...] = jnp.maximum(
        jnp.dot(t2_ref[...], wf1_ref[...], preferred_element_type=f32)
        + bf1_ref[...], 0.0)
    out_ref[...] = jnp.dot(h_ref[...], wf2_ref[...],
                           preferred_element_type=f32) + bf2_ref[...]


def _lsm_kernel(z_ref, o_ref):
    z = z_ref[...]
    mx = jnp.max(z, axis=0, keepdims=True)
    lse = jnp.log(jnp.sum(jnp.exp(z - mx), axis=0, keepdims=True)) + mx
    o_ref[...] = z - lse


def _round_up(a, b):
    return (a + b - 1) // b * b


@jax.jit
def _forward(x, w1, b1, w2, b2, d1, wf1x, bf1, wf2, bf2):
    f32 = jnp.float32
    bf16 = jnp.bfloat16
    B = x.shape[0]
    xr = x.astype(f32).reshape(B, IMG * IMG)

    w1m, w2m = _build_band_mats(w1, w2)
    w1m = w1m.astype(bf16)
    w2m = w2m.astype(bf16)
    d1h = d1.astype(bf16)
    b1r = jnp.repeat(b1, S1).reshape(1, N1)
    b2r = jnp.repeat(b2, S3).reshape(1, N2)
    wf1 = jnp.pad(wf1x, ((0, 0), (0, S3 - M2), (0, 0))).reshape(N2, FCH)
    wf1 = wf1.astype(bf16)

    bt = min(_round_up(B, 8), BT)
    b_pad = _round_up(B, bt)
    if b_pad != B:
        xr = jnp.pad(xr, ((0, b_pad - B), (0, 0)))

    vmem = pl.BlockSpec(memory_space=pltpu.MemorySpace.VMEM)

    logits = pl.pallas_call(
        _fwd_kernel,
        out_shape=jax.ShapeDtypeStruct((b_pad, NCLS), f32),
        grid=(b_pad // bt,),
        in_specs=[
            pl.BlockSpec((bt, IMG * IMG), lambda i: (i, 0)),
            vmem, vmem, vmem, vmem,
            vmem,
            vmem, vmem, vmem, vmem,
        ],
        out_specs=pl.BlockSpec((bt, NCLS), lambda i: (i, 0)),
        scratch_shapes=[
            pltpu.VMEM((bt, PW1 * PW1), bf16),
            pltpu.VMEM((bt, N1), f32),
            pltpu.VMEM((bt, N1), bf16),
            pltpu.VMEM((bt, K2), bf16),
            pltpu.VMEM((bt, N2), f32),
            pltpu.VMEM((bt, N2), bf16),
            pltpu.VMEM((bt, FCH), f32),
        ],
        compiler_params=pltpu.CompilerParams(
            dimension_semantics=("parallel",)),
    )(xr, w1m, b1r, w2m, b2r, d1h, wf1, bf1, wf2, bf2)

    logits = logits[:B]

    return pl.pallas_call(
        _lsm_kernel,
        out_shape=jax.ShapeDtypeStruct((B, NCLS), f32),
        in_specs=[vmem],
        out_specs=vmem,
    )(logits)


def kernel(x, w1, b1, w2, b2, d1, wf1x, bf1, wf2, bf2):
    return _forward(x, w1, b1, w2, b2, d1, wf1x, bf1, wf2, bf2)


# speedup vs baseline: 6.8850x; 1.0872x over previous
import numpy as np
import jax
import jax.numpy as jnp
from jax.experimental import pallas as pl
from jax.experimental.pallas import tpu as pltpu

IMG = 28
C1, C2 = 4, 8
NCLS = 10
PW1 = 32
H2 = 14
H3 = 7
PLANE1 = 256
SLOT1 = 4 * PLANE1
N1 = C1 * SLOT1
POOL1 = 256
K2 = C1 * POOL1
PLANE2 = 64
SLOT2 = 4 * PLANE2
N2 = C2 * SLOT2
POOL2 = 64
KF = C2 * POOL2
FCH = 512
BT = 256


def _band1(nq, k, e, n):
    q = np.arange(nq)[:, None]
    f = np.arange(n)[None, :]
    return ((q == 2 * f + e + k) & (q >= 2) & (q <= 29)).astype(np.float32)


def _band2(k, e):
    x = np.arange(H2)[:, None]
    f = np.arange(H3)[None, :]
    return (x == 2 * f + e + k - 2).astype(np.float32)


_U1 = np.stack([np.stack([_band1(PW1, k, e, H2) for e in range(2)], 1)
                for k in range(5)])
_U2 = np.stack([np.stack([_band2(k, e) for e in range(2)], 1)
                for k in range(5)])
_ROWS49 = np.array([2 * i * 18 + 2 * j for i in range(H3) for j in range(H3)],
                   np.int32)


def _build_mats(w1, w2, wf1x):
    f32 = jnp.float32
    bf16 = jnp.bfloat16
    u1 = jnp.asarray(_U1)
    u2 = jnp.asarray(_U2)

    v1 = jnp.einsum('ckj,jrgh->ckrgh', w1.reshape(C1, 5, 5), u1)
    w1m = jnp.einsum('kqef,ckrgh->qrcegfh', u1, v1,
                     preferred_element_type=f32)
    w1m = w1m.reshape(PW1 * PW1, C1, 4, H2 * H2)
    w1m = jnp.pad(w1m, ((0, 0), (0, 0), (0, 0), (0, PLANE1 - H2 * H2)))
    w1m = w1m.reshape(PW1 * PW1, N1).astype(bf16)

    v2 = jnp.einsum('ockj,jygh->ockygh', w2.reshape(C2, C1, 5, 5), u2)
    w2m = jnp.einsum('kxef,ockygh->cxyoegfh', u2, v2,
                     preferred_element_type=f32)
    w2m = w2m.reshape(C1, H2 * H2, C2, 4, H3 * H3)
    w2m = jnp.pad(w2m, ((0, 0), (0, POOL1 - H2 * H2),
                        (0, 0), (0, 0), (0, PLANE2 - H3 * H3)))
    w2m = w2m.reshape(C1 * POOL1, N2).astype(bf16)

    wf1c = wf1x[:, _ROWS49, :]
    wf1c = jnp.pad(wf1c, ((0, 0), (0, POOL2 - H3 * H3), (0, 0)))
    wf1c = wf1c.reshape(KF, FCH).astype(bf16)
    return w1m, w2m, wf1c


def _fwd_kernel(xr_ref, w1m_ref, b1r_ref, w2m_ref, b2r_ref,
                wf1_ref, bf1_ref, wf2_ref, bf2_ref, out_ref,
                xs_ref, y1_ref, xp2_ref, y2_ref, t2_ref, h_ref):
    f32 = jnp.float32
    bf16 = jnp.bfloat16
    bt = xr_ref.shape[0]

    xs_ref[...] = jnp.zeros((bt, PW1 * PW1), bf16)
    for i in range(IMG):
        dst = (i + 2) * PW1 + 2
        xs_ref[:, dst:dst + IMG] = xr_ref[:, i * IMG:(i + 1) * IMG].astype(bf16)

    y1_ref[...] = jnp.maximum(
        jnp.dot(xs_ref[...], w1m_ref[...], preferred_element_type=f32)
        + b1r_ref[...], 0.0).astype(bf16)

    for ci in range(C1):
        b = ci * SLOT1
        n = H2 * H2
        xp2_ref[:, ci * POOL1:ci * POOL1 + n] = jnp.maximum(
            jnp.maximum(y1_ref[:, b:b + n],
                        y1_ref[:, b + PLANE1:b + PLANE1 + n]),
            jnp.maximum(y1_ref[:, b + 2 * PLANE1:b + 2 * PLANE1 + n],
                        y1_ref[:, b + 3 * PLANE1:b + 3 * PLANE1 + n]))
        xp2_ref[:, ci * POOL1 + n:(ci + 1) * POOL1] = jnp.zeros(
            (bt, POOL1 - n), bf16)

    y2_ref[...] = jnp.maximum(
        jnp.dot(xp2_ref[...], w2m_ref[...], preferred_element_type=f32)
        + b2r_ref[...], 0.0).astype(bf16)

    for co in range(C2):
        b = co * SLOT2
        n = H3 * H3
        t2_ref[:, co * POOL2:co * POOL2 + n] = jnp.maximum(
            jnp.maximum(y2_ref[:, b:b + n],
                        y2_ref[:, b + PLANE2:b + PLANE2 + n]),
            jnp.maximum(y2_ref[:, b + 2 * PLANE2:b + 2 * PLANE2 + n],
                        y2_ref[:, b + 3 * PLANE2:b + 3 * PLANE2 + n]))
        t2_ref[:, co * POOL2 + n:(co + 1) * POOL2] = jnp.zeros(
            (bt, POOL2 - n), bf16)

    h_ref[...] = jnp.maximum(
        jnp.dot(t2_ref[...], wf1_ref[...], preferred_element_type=f32)
        + bf1_ref[...], 0.0)
    out_ref[...] = jnp.dot(h_ref[...], wf2_ref[...],
                           preferred_element_type=f32) + bf2_ref[...]


def _lsm_kernel(z_ref, o_ref):
    z = z_ref[...]
    mx = jnp.max(z, axis=0, keepdims=True)
    lse = jnp.log(jnp.sum(jnp.exp(z - mx), axis=0, keepdims=True)) + mx
    o_ref[...] = z - lse


def _round_up(a, b):
    return (a + b - 1) // b * b


@jax.jit
def _forward(x, w1, b1, w2, b2, d1, wf1x, bf1, wf2, bf2):
    del d1
    f32 = jnp.float32
    B = x.shape[0]
    xr = x.astype(f32).reshape(B, IMG * IMG)

    w1m, w2m, wf1c = _build_mats(w1, w2, wf1x)
    b1r = jnp.repeat(b1, SLOT1).reshape(1, N1)
    b2r = jnp.repeat(b2, SLOT2).reshape(1, N2)

    bt = min(_round_up(B, 8), BT)
    b_pad = _round_up(B, bt)
    if b_pad != B:
        xr = jnp.pad(xr, ((0, b_pad - B), (0, 0)))

    vmem = pl.BlockSpec(memory_space=pltpu.MemorySpace.VMEM)

    logits = pl.pallas_call(
        _fwd_kernel,
        out_shape=jax.ShapeDtypeStruct((b_pad, NCLS), f32),
        grid=(b_pad // bt,),
        in_specs=[
            pl.BlockSpec((bt, IMG * IMG), lambda i: (i, 0)),
            vmem, vmem, vmem, vmem,
            vmem, vmem, vmem, vmem,
        ],
        out_specs=pl.BlockSpec((bt, NCLS), lambda i: (i, 0)),
        scratch_shapes=[
            pltpu.VMEM((bt, PW1 * PW1), jnp.bfloat16),
            pltpu.VMEM((bt, N1), jnp.bfloat16),
            pltpu.VMEM((bt, K2), jnp.bfloat16),
            pltpu.VMEM((bt, N2), jnp.bfloat16),
            pltpu.VMEM((bt, KF), jnp.bfloat16),
            pltpu.VMEM((bt, FCH), f32),
        ],
        compiler_params=pltpu.CompilerParams(
            dimension_semantics=("parallel",)),
    )(xr, w1m, b1r, w2m, b2r, wf1c, bf1, wf2, bf2)

    logits = logits[:B]

    return pl.pallas_call(
        _lsm_kernel,
        out_shape=jax.ShapeDtypeStruct((B, NCLS), f32),
        in_specs=[vmem],
        out_specs=vmem,
    )(logits)


def kernel(x, w1, b1, w2, b2, d1, wf1x, bf1, wf2, bf2):
    return _forward(x, w1, b1, w2, b2, d1, wf1x, bf1, wf2, bf2)
